# baseline (speedup 1.0000x reference)
import jax
import jax.numpy as jnp
from jax import lax
from jax.experimental import pallas as pl
from jax.experimental.pallas import tpu as pltpu
from jax.experimental.pallas import tpu_sc as plsc

NU = 10000
NI = 10000
E = 160000
F = 256
HF = 128
NC = 2
NS = 16

K1 = 40
NCHUNK1 = (E // NS) // K1
K2 = 120
NCHUNK2 = (3 * E // (NC * NS)) // K2

PAD1 = 10240
PAD2 = 30720
ROWS_PER_TILE1 = PAD1 // NS
ROWS_PER_TILE2 = PAD2 // NS


def _stage_a_body(x_ref, w1f_ref, b1f_ref, w1b_ref, b1b_ref, tf_ref, tb_ref):
    x = x_ref[...]
    whf = jnp.dot(x, w1f_ref[...], preferred_element_type=jnp.float32) + b1f_ref[...]
    whb = jnp.dot(x, w1b_ref[...], preferred_element_type=jnp.float32) + b1b_ref[...]
    tf_ref[0, ...] = whf[:, :HF]
    tf_ref[1, ...] = whf[:, HF:]
    tb_ref[0, ...] = whb[:, :HF]
    tb_ref[1, ...] = whb[:, HF:]


def _stage_a(x, w1f, b1f, w1b, b1b):
    blk = 2000
    grid = NU // blk
    return pl.pallas_call(
        _stage_a_body,
        grid=(grid,),
        in_specs=[
            pl.BlockSpec((blk, F), lambda i: (i, 0)),
            pl.BlockSpec((F, F), lambda i: (0, 0)),
            pl.BlockSpec((1, F), lambda i: (0, 0)),
            pl.BlockSpec((F, F), lambda i: (0, 0)),
            pl.BlockSpec((1, F), lambda i: (0, 0)),
        ],
        out_specs=[
            pl.BlockSpec((NC, blk, HF), lambda i: (0, i, 0)),
            pl.BlockSpec((NC, blk, HF), lambda i: (0, i, 0)),
        ],
        out_shape=[
            jax.ShapeDtypeStruct((NC, NU, HF), jnp.float32),
            jax.ShapeDtypeStruct((NC, NU, HF), jnp.float32),
        ],
    )(x, w1f, b1f.reshape(1, F), w1b, b1b.reshape(1, F))


def _edge_loop(tab, idx_src, idx_dst, rowsbuf, gsems, ssems, acc, nchunks,
               cnt_scatters):
    def g_issue(c, p):
        pltpu.async_copy(tab.at[idx_src.at[c]], rowsbuf.at[p], gsems[p])

    def g_wait(p):
        pltpu.make_async_copy(tab.at[idx_src.at[0]], rowsbuf.at[p],
                              gsems[p]).wait()

    def s_issue(c, p):
        pltpu.async_copy(rowsbuf.at[p], acc.at[idx_dst.at[c]], ssems[p],
                         add=True)

    def s_wait(p):
        pltpu.make_async_copy(rowsbuf.at[p], acc.at[idx_dst.at[0]],
                              ssems[p]).wait()

    def counts(c):
        for pred, ones_ref, cnt_acc, cnt_idx in cnt_scatters:
            @pl.when(pred)
            def _():
                pltpu.sync_copy(ones_ref, cnt_acc.at[cnt_idx.at[c]], add=True)

    g_issue(0, 0)
    g_wait(0)
    s_issue(0, 0)
    counts(0)

    @pl.when(1 < nchunks)
    def _():
        g_issue(1, 1)

    def body(h, _):
        c1 = 2 * h + 1
        c2 = 2 * h + 2

        @pl.when(c1 < nchunks)
        def _():
            g_wait(1)
            s_issue(c1, 1)
            counts(c1)
            s_wait(0)

            @pl.when(c2 < nchunks)
            def _():
                g_issue(c2, 0)

        @pl.when(c2 < nchunks)
        def _():
            g_wait(0)
            s_issue(c2, 0)
            counts(c2)
            s_wait(1)

            @pl.when(c2 + 1 < nchunks)
            def _():
                g_issue(c2 + 1, 1)

        return 0

    lax.fori_loop(0, nchunks // 2, body, 0)
    s_wait((nchunks - 1) % 2)


def _sc1_body(tf_hbm, tb_hbm, srcf_hbm, dstf_hbm, srcb_hbm, dstb_hbm,
              dstbb_hbm, z128_hbm, z16_hbm, ones_hbm,
              sf_out, sb_out, cntf_out, cntb_out, cntbb_out,
              acc, cnt, idx_src, idx_dst, rows, ones_v,
              gsem0, gsem1, ssem0, ssem1):
    cid = lax.axis_index("c")
    sid = lax.axis_index("s")
    row0 = sid * ROWS_PER_TILE1

    pltpu.sync_copy(ones_hbm, ones_v)

    def run_phase(tab, src2, dst, s_out, cnt_zero_pred, cnt_scatters,
                  cnt_flushes):
        pltpu.sync_copy(z128_hbm, acc.at[pl.ds(row0, ROWS_PER_TILE1)])

        @pl.when(cnt_zero_pred)
        def _():
            pltpu.sync_copy(z16_hbm, cnt.at[pl.ds(row0, ROWS_PER_TILE1)])

        plsc.subcore_barrier()

        pltpu.sync_copy(src2.at[cid, sid], idx_src)
        pltpu.sync_copy(dst.at[sid], idx_dst)

        _edge_loop(tab, idx_src, idx_dst, rows, (gsem0, gsem1),
                   (ssem0, ssem1), acc, NCHUNK1, cnt_scatters)
        plsc.subcore_barrier()

        pltpu.sync_copy(acc.at[pl.ds(row0, ROWS_PER_TILE1)],
                        s_out.at[cid, pl.ds(row0, ROWS_PER_TILE1)])

        for pred, cnt_out in cnt_flushes:
            @pl.when(pred)
            def _():
                pltpu.sync_copy(cnt.at[pl.ds(row0, ROWS_PER_TILE1)],
                                cnt_out.at[pl.ds(row0, ROWS_PER_TILE1)])

    run_phase(tf_hbm, srcf_hbm, dstf_hbm, sf_out,
              cnt_zero_pred=(cid >= 0),
              cnt_scatters=[(cid == 0, ones_v, cnt, idx_dst)],
              cnt_flushes=[(cid == 0, cntf_out)])

    @pl.when(cid == 1)
    def _():
        pltpu.sync_copy(dstbb_hbm.at[sid], idx_dst)

        def bb_group(h, _):
            for j in range(25):
                pltpu.async_copy(ones_v, cnt.at[idx_dst.at[h * 25 + j]],
                                 ssem0, add=True)
            for j in range(25):
                pltpu.make_async_copy(ones_v, cnt.at[idx_dst.at[0]],
                                      ssem0).wait()
            return 0

        lax.fori_loop(0, NCHUNK1 // 25, bb_group, 0)

    plsc.subcore_barrier()

    @pl.when(cid == 1)
    def _():
        pltpu.sync_copy(cnt.at[pl.ds(row0, ROWS_PER_TILE1)],
                        cntbb_out.at[pl.ds(row0, ROWS_PER_TILE1)])

    run_phase(tb_hbm, srcb_hbm, dstb_hbm, sb_out,
              cnt_zero_pred=(cid == 0),
              cnt_scatters=[(cid == 0, ones_v, cnt, idx_dst)],
              cnt_flushes=[(cid == 0, cntb_out)])


def _stage_b(tf, tb, srcf2, dstf, srcb2, dstb, dstbb, z128, z16, ones16):
    mesh = plsc.VectorSubcoreMesh(core_axis_name="c", subcore_axis_name="s",
                                  num_cores=NC, num_subcores=NS)
    f = pl.kernel(
        _sc1_body,
        out_type=[
            jax.ShapeDtypeStruct((NC, PAD1, HF), jnp.float32),
            jax.ShapeDtypeStruct((NC, PAD1, HF), jnp.float32),
            jax.ShapeDtypeStruct((PAD1, 16), jnp.float32),
            jax.ShapeDtypeStruct((PAD1, 16), jnp.float32),
            jax.ShapeDtypeStruct((PAD1, 16), jnp.float32),
        ],
        mesh=mesh,
        scratch_types=[
            pltpu.VMEM_SHARED((PAD1, HF), jnp.float32),
            pltpu.VMEM_SHARED((PAD1, 16), jnp.float32),
            pltpu.VMEM((NCHUNK1, K1), jnp.int32),
            pltpu.VMEM((NCHUNK1, K1), jnp.int32),
            pltpu.VMEM((2, K1, HF), jnp.float32),
            pltpu.VMEM((K1, 16), jnp.float32),
            pltpu.SemaphoreType.DMA,
            pltpu.SemaphoreType.DMA,
            pltpu.SemaphoreType.DMA,
            pltpu.SemaphoreType.DMA,
        ],
        compiler_params=pltpu.CompilerParams(use_tc_tiling_on_sc=False),
    )
    return f(tf, tb, srcf2, dstf, srcb2, dstb, dstbb, z128, z16, ones16)


def _stage_c_body(sf_ref, sb_ref, cntf_ref, cntb_ref, cntbb_ref, b1bb_ref,
                  w2f_ref, b2f_ref, w2bb_ref, b2bb_ref, w2b_ref, b2b_ref,
                  h2u_ref, h2i_ref, t2_ref):
    cf = jnp.maximum(cntf_ref[:, 0:1], 1.0)
    cb = jnp.maximum(cntb_ref[:, 0:1], 1.0)
    ibb = (cntbb_ref[:, 0:1] > 0.0).astype(jnp.float32)
    h1u = jnp.concatenate([sf_ref[0], sf_ref[1]], axis=1) / cf + b1bb_ref[...] * ibb
    h1i = jnp.concatenate([sb_ref[0], sb_ref[1]], axis=1) / cb
    h2u = jnp.where(h1u >= 0.0, h1u, 0.01 * h1u)
    h2i = jnp.where(h1i >= 0.0, h1i, 0.01 * h1i)
    h2u_ref[...] = h2u
    h2i_ref[...] = h2i
    t2_ref[0, ...] = jnp.dot(h2u, w2f_ref[...], preferred_element_type=jnp.float32) + b2f_ref[...]
    t2_ref[1, ...] = jnp.dot(h2i, w2bb_ref[...], preferred_element_type=jnp.float32) + b2bb_ref[...]
    t2_ref[2, ...] = jnp.dot(h2u, w2b_ref[...], preferred_element_type=jnp.float32) + b2b_ref[...]


def _stage_c(sf, sb, cntf, cntb, cntbb, b1bb, w2f_p, b2f_p, w2bb_p, b2bb_p,
             w2b_p, b2b_p):
    blk = 2000
    grid = NU // blk
    return pl.pallas_call(
        _stage_c_body,
        grid=(grid,),
        in_specs=[
            pl.BlockSpec((NC, blk, HF), lambda i: (0, i, 0)),
            pl.BlockSpec((NC, blk, HF), lambda i: (0, i, 0)),
            pl.BlockSpec((blk, 16), lambda i: (i, 0)),
            pl.BlockSpec((blk, 16), lambda i: (i, 0)),
            pl.BlockSpec((blk, 16), lambda i: (i, 0)),
            pl.BlockSpec((1, F), lambda i: (0, 0)),
            pl.BlockSpec((F, 16), lambda i: (0, 0)),
            pl.BlockSpec((1, 16), lambda i: (0, 0)),
            pl.BlockSpec((F, 16), lambda i: (0, 0)),
            pl.BlockSpec((1, 16), lambda i: (0, 0)),
            pl.BlockSpec((F, 16), lambda i: (0, 0)),
            pl.BlockSpec((1, 16), lambda i: (0, 0)),
        ],
        out_specs=[
            pl.BlockSpec((blk, F), lambda i: (i, 0)),
            pl.BlockSpec((blk, F), lambda i: (i, 0)),
            pl.BlockSpec((3, blk, 16), lambda i: (0, i, 0)),
        ],
        out_shape=[
            jax.ShapeDtypeStruct((NU, F), jnp.float32),
            jax.ShapeDtypeStruct((NI, F), jnp.float32),
            jax.ShapeDtypeStruct((3, NU, 16), jnp.float32),
        ],
    )(sf, sb, cntf, cntb, cntbb, b1bb, w2f_p, b2f_p, w2bb_p, b2bb_p, w2b_p, b2b_p)


def _sc2_body(t2_hbm, src_hbm, dst_hbm, z16_hbm,
              s2_out, acc, idx_src, idx_dst, rows,
              gsem0, gsem1, ssem0, ssem1):
    cid = lax.axis_index("c")
    sid = lax.axis_index("s")
    row0 = sid * ROWS_PER_TILE2

    for q in range(3):
        pltpu.sync_copy(z16_hbm, acc.at[pl.ds(row0 + q * 640, 640)])
    plsc.subcore_barrier()

    pltpu.sync_copy(src_hbm.at[cid, sid], idx_src)
    pltpu.sync_copy(dst_hbm.at[cid, sid], idx_dst)

    _edge_loop(t2_hbm, idx_src, idx_dst, rows, (gsem0, gsem1),
               (ssem0, ssem1), acc, NCHUNK2, [])
    plsc.subcore_barrier()

    pltpu.sync_copy(acc.at[pl.ds(row0, ROWS_PER_TILE2)],
                    s2_out.at[cid, pl.ds(row0, ROWS_PER_TILE2)])


def _stage_d(t2, bsrc, bdst, z16):
    mesh = plsc.VectorSubcoreMesh(core_axis_name="c", subcore_axis_name="s",
                                  num_cores=NC, num_subcores=NS)
    f = pl.kernel(
        _sc2_body,
        out_type=jax.ShapeDtypeStruct((NC, PAD2, 16), jnp.float32),
        mesh=mesh,
        scratch_types=[
            pltpu.VMEM_SHARED((PAD2, 16), jnp.float32),
            pltpu.VMEM((NCHUNK2, K2), jnp.int32),
            pltpu.VMEM((NCHUNK2, K2), jnp.int32),
            pltpu.VMEM((2, K2, 16), jnp.float32),
            pltpu.SemaphoreType.DMA,
            pltpu.SemaphoreType.DMA,
            pltpu.SemaphoreType.DMA,
            pltpu.SemaphoreType.DMA,
        ],
        compiler_params=pltpu.CompilerParams(use_tc_tiling_on_sc=False),
    )
    return f(t2, bsrc, bdst, z16)


def _stage_e_body(s2_ref, cntf_ref, cntb_ref, cntbb_ref, ou_ref, oi_ref):
    sf = s2_ref[0, 0] + s2_ref[1, 0]
    sbb = s2_ref[0, 1] + s2_ref[1, 1]
    sb = s2_ref[0, 2] + s2_ref[1, 2]
    ou_ref[...] = (sf / jnp.maximum(cntf_ref[:, 0:1], 1.0)
                   + sbb / jnp.maximum(cntbb_ref[:, 0:1], 1.0))
    oi_ref[...] = sb / jnp.maximum(cntb_ref[:, 0:1], 1.0)


def _stage_e(s2, cntf, cntb, cntbb):
    blk = 2000
    grid = NU // blk
    return pl.pallas_call(
        _stage_e_body,
        grid=(grid,),
        in_specs=[
            pl.BlockSpec((NC, 3, blk, 16), lambda i: (0, 0, i, 0)),
            pl.BlockSpec((blk, 16), lambda i: (i, 0)),
            pl.BlockSpec((blk, 16), lambda i: (i, 0)),
            pl.BlockSpec((blk, 16), lambda i: (i, 0)),
        ],
        out_specs=[
            pl.BlockSpec((blk, 16), lambda i: (i, 0)),
            pl.BlockSpec((blk, 16), lambda i: (i, 0)),
        ],
        out_shape=[
            jax.ShapeDtypeStruct((NU, 16), jnp.float32),
            jax.ShapeDtypeStruct((NI, 16), jnp.float32),
        ],
    )(s2, cntf, cntb, cntbb)


@jax.jit
def kernel(node_feature, follows_src, follows_dst, buys_src, buys_dst,
           bought_by_src, bought_by_dst,
           W1_follows, b1_follows, W1_buys, b1_buys, W1_bought_by, b1_bought_by,
           W2_follows, b2_follows, W2_buys, b2_buys, W2_bought_by, b2_bought_by):
    srcf2 = jnp.stack([follows_src, follows_src + NU]).reshape(NC, NS, NCHUNK1, K1)
    srcb2 = jnp.stack([buys_src, buys_src + NU]).reshape(NC, NS, NCHUNK1, K1)
    dstf = follows_dst.reshape(NS, NCHUNK1, K1)
    dstb = buys_dst.reshape(NS, NCHUNK1, K1)
    dstbb = bought_by_dst.reshape(NS, NCHUNK1, K1)
    bsrc = jnp.concatenate([follows_src, bought_by_src + NU,
                            buys_src + 2 * NU]).reshape(NC, NS, NCHUNK2, K2)
    bdst = jnp.concatenate([follows_dst, bought_by_dst + NU,
                            buys_dst + 2 * NU]).reshape(NC, NS, NCHUNK2, K2)
    z128 = jnp.zeros((ROWS_PER_TILE1, HF), jnp.float32)
    z16 = jnp.zeros((640, 16), jnp.float32)
    ones16 = jnp.ones((K1, 16), jnp.float32)

    def pad16(w, b):
        wp = jnp.zeros((F, 16), jnp.float32).at[:, :2].set(w)
        bp = jnp.zeros((1, 16), jnp.float32).at[0, :2].set(b)
        return wp, bp

    w2f_p, b2f_p = pad16(W2_follows, b2_follows)
    w2bb_p, b2bb_p = pad16(W2_bought_by, b2_bought_by)
    w2b_p, b2b_p = pad16(W2_buys, b2_buys)

    tf, tb = _stage_a(node_feature, W1_follows, b1_follows, W1_buys, b1_buys)

    sf, sb, cntf, cntb, cntbb = _stage_b(
        tf.reshape(NC * NU, HF), tb.reshape(NC * NI, HF),
        srcf2, dstf, srcb2, dstb, dstbb, z128, z16, ones16)

    sf = sf[:, :NU]
    sb = sb[:, :NU]
    cntf = cntf[:NU]
    cntb = cntb[:NU]
    cntbb = cntbb[:NU]

    h2u, h2i, t2 = _stage_c(sf, sb, cntf, cntb, cntbb,
                            b1_bought_by.reshape(1, F),
                            w2f_p, b2f_p, w2bb_p, b2bb_p, w2b_p, b2b_p)

    s2 = _stage_d(t2.reshape(3 * NU, 16), bsrc, bdst, z16)

    ou, oi = _stage_e(s2[:, :3 * NU].reshape(NC, 3, NU, 16), cntf, cntb, cntbb)

    return ou[:, :2], oi[:, :2], h2u, h2i


# speedup vs baseline: 1.3554x; 1.3554x over previous
import jax
import jax.numpy as jnp
from jax import lax
from jax.experimental import pallas as pl
from jax.experimental.pallas import tpu as pltpu
from jax.experimental.pallas import tpu_sc as plsc

NU = 10000
NI = 10000
E = 160000
F = 256
HF = 128
NC = 2
NS = 16

K1 = 40
NCHUNK1 = (E // NS) // K1
K2 = 120
NCHUNK2 = (3 * E // (NC * NS)) // K2

PAD1 = 10240
PAD2 = 30720
ROWS_PER_TILE1 = PAD1 // NS
ROWS_PER_TILE2 = PAD2 // NS


def _stage_a_body(x_ref, w1f_ref, b1f_ref, w1b_ref, b1b_ref, tf_ref, tb_ref):
    x = x_ref[...]
    whf = jnp.dot(x, w1f_ref[...], preferred_element_type=jnp.float32) + b1f_ref[...]
    whb = jnp.dot(x, w1b_ref[...], preferred_element_type=jnp.float32) + b1b_ref[...]
    tf_ref[0, ...] = whf[:, :HF]
    tf_ref[1, ...] = whf[:, HF:]
    tb_ref[0, ...] = whb[:, :HF]
    tb_ref[1, ...] = whb[:, HF:]


def _stage_a(x, w1f, b1f, w1b, b1b):
    blk = 2000
    grid = NU // blk
    return pl.pallas_call(
        _stage_a_body,
        grid=(grid,),
        in_specs=[
            pl.BlockSpec((blk, F), lambda i: (i, 0)),
            pl.BlockSpec((F, F), lambda i: (0, 0)),
            pl.BlockSpec((1, F), lambda i: (0, 0)),
            pl.BlockSpec((F, F), lambda i: (0, 0)),
            pl.BlockSpec((1, F), lambda i: (0, 0)),
        ],
        out_specs=[
            pl.BlockSpec((NC, blk, HF), lambda i: (0, i, 0)),
            pl.BlockSpec((NC, blk, HF), lambda i: (0, i, 0)),
        ],
        out_shape=[
            jax.ShapeDtypeStruct((NC, NU, HF), jnp.float32),
            jax.ShapeDtypeStruct((NC, NU, HF), jnp.float32),
        ],
    )(x, w1f, b1f.reshape(1, F), w1b, b1b.reshape(1, F))


def _edge_loop(tab, idx_src, idx_dst, rowsbuf, gsems, ssems, acc, nchunks,
               cnt_scatters):
    del ssems

    def g_issue(c, p):
        pltpu.async_copy(tab.at[idx_src.at[c]], rowsbuf.at[p], gsems[p])

    def g_wait(p):
        pltpu.make_async_copy(tab.at[idx_src.at[0]], rowsbuf.at[p],
                              gsems[p]).wait()

    def consume(c, p):
        pltpu.sync_copy(rowsbuf.at[p], acc.at[idx_dst.at[c]], add=True)
        for pred, ones_ref, cnt_acc, cnt_idx in cnt_scatters:
            @pl.when(pred)
            def _():
                pltpu.sync_copy(ones_ref, cnt_acc.at[cnt_idx.at[c]], add=True)

    g_issue(0, 0)

    def body(h, _):
        c0 = 2 * h
        c1 = 2 * h + 1

        @pl.when(c1 < nchunks)
        def _():
            g_issue(c1, 1)

        g_wait(0)
        consume(c0, 0)

        @pl.when(c0 + 2 < nchunks)
        def _():
            g_issue(c0 + 2, 0)

        @pl.when(c1 < nchunks)
        def _():
            g_wait(1)
            consume(c1, 1)

        return 0

    lax.fori_loop(0, (nchunks + 1) // 2, body, 0)


def _sc1_body(tf_hbm, tb_hbm, srcf_hbm, dstf_hbm, srcb_hbm, dstb_hbm,
              dstbb_hbm, z128_hbm, z16_hbm, ones_hbm,
              sf_out, sb_out, cntf_out, cntb_out, cntbb_out,
              acc, cnt, idx_src, idx_dst, rows, ones_v,
              gsem0, gsem1, ssem0, ssem1):
    cid = lax.axis_index("c")
    sid = lax.axis_index("s")
    row0 = sid * ROWS_PER_TILE1

    pltpu.sync_copy(ones_hbm, ones_v)

    def run_phase(tab, src2, dst, s_out, cnt_zero_pred, cnt_scatters,
                  cnt_flushes):
        pltpu.sync_copy(z128_hbm, acc.at[pl.ds(row0, ROWS_PER_TILE1)])

        @pl.when(cnt_zero_pred)
        def _():
            pltpu.sync_copy(z16_hbm, cnt.at[pl.ds(row0, ROWS_PER_TILE1)])

        plsc.subcore_barrier()

        pltpu.sync_copy(src2.at[cid, sid], idx_src)
        pltpu.sync_copy(dst.at[sid], idx_dst)

        _edge_loop(tab, idx_src, idx_dst, rows, (gsem0, gsem1),
                   (ssem0, ssem1), acc, NCHUNK1, cnt_scatters)
        plsc.subcore_barrier()

        pltpu.sync_copy(acc.at[pl.ds(row0, ROWS_PER_TILE1)],
                        s_out.at[cid, pl.ds(row0, ROWS_PER_TILE1)])

        for pred, cnt_out in cnt_flushes:
            @pl.when(pred)
            def _():
                pltpu.sync_copy(cnt.at[pl.ds(row0, ROWS_PER_TILE1)],
                                cnt_out.at[pl.ds(row0, ROWS_PER_TILE1)])

    run_phase(tf_hbm, srcf_hbm, dstf_hbm, sf_out,
              cnt_zero_pred=(cid >= 0),
              cnt_scatters=[(cid == 0, ones_v, cnt, idx_dst)],
              cnt_flushes=[(cid == 0, cntf_out)])

    @pl.when(cid == 1)
    def _():
        pltpu.sync_copy(dstbb_hbm.at[sid], idx_dst)

        def bb_group(h, _):
            for j in range(25):
                pltpu.async_copy(ones_v, cnt.at[idx_dst.at[h * 25 + j]],
                                 ssem0, add=True)
            for j in range(25):
                pltpu.make_async_copy(ones_v, cnt.at[idx_dst.at[0]],
                                      ssem0).wait()
            return 0

        lax.fori_loop(0, NCHUNK1 // 25, bb_group, 0)

    plsc.subcore_barrier()

    @pl.when(cid == 1)
    def _():
        pltpu.sync_copy(cnt.at[pl.ds(row0, ROWS_PER_TILE1)],
                        cntbb_out.at[pl.ds(row0, ROWS_PER_TILE1)])

    run_phase(tb_hbm, srcb_hbm, dstb_hbm, sb_out,
              cnt_zero_pred=(cid == 0),
              cnt_scatters=[(cid == 0, ones_v, cnt, idx_dst)],
              cnt_flushes=[(cid == 0, cntb_out)])


def _stage_b(tf, tb, srcf2, dstf, srcb2, dstb, dstbb, z128, z16, ones16):
    mesh = plsc.VectorSubcoreMesh(core_axis_name="c", subcore_axis_name="s",
                                  num_cores=NC, num_subcores=NS)
    f = pl.kernel(
        _sc1_body,
        out_type=[
            jax.ShapeDtypeStruct((NC, PAD1, HF), jnp.float32),
            jax.ShapeDtypeStruct((NC, PAD1, HF), jnp.float32),
            jax.ShapeDtypeStruct((PAD1, 16), jnp.float32),
            jax.ShapeDtypeStruct((PAD1, 16), jnp.float32),
            jax.ShapeDtypeStruct((PAD1, 16), jnp.float32),
        ],
        mesh=mesh,
        scratch_types=[
            pltpu.VMEM_SHARED((PAD1, HF), jnp.float32),
            pltpu.VMEM_SHARED((PAD1, 16), jnp.float32),
            pltpu.VMEM((NCHUNK1, K1), jnp.int32),
            pltpu.VMEM((NCHUNK1, K1), jnp.int32),
            pltpu.VMEM((2, K1, HF), jnp.float32),
            pltpu.VMEM((K1, 16), jnp.float32),
            pltpu.SemaphoreType.DMA,
            pltpu.SemaphoreType.DMA,
            pltpu.SemaphoreType.DMA,
            pltpu.SemaphoreType.DMA,
        ],
        compiler_params=pltpu.CompilerParams(use_tc_tiling_on_sc=False),
    )
    return f(tf, tb, srcf2, dstf, srcb2, dstb, dstbb, z128, z16, ones16)


def _stage_c_body(sf_ref, sb_ref, cntf_ref, cntb_ref, cntbb_ref, b1bb_ref,
                  w2f_ref, b2f_ref, w2bb_ref, b2bb_ref, w2b_ref, b2b_ref,
                  h2u_ref, h2i_ref, t2_ref):
    cf = jnp.maximum(cntf_ref[:, 0:1], 1.0)
    cb = jnp.maximum(cntb_ref[:, 0:1], 1.0)
    ibb = (cntbb_ref[:, 0:1] > 0.0).astype(jnp.float32)
    h1u = jnp.concatenate([sf_ref[0], sf_ref[1]], axis=1) / cf + b1bb_ref[...] * ibb
    h1i = jnp.concatenate([sb_ref[0], sb_ref[1]], axis=1) / cb
    h2u = jnp.where(h1u >= 0.0, h1u, 0.01 * h1u)
    h2i = jnp.where(h1i >= 0.0, h1i, 0.01 * h1i)
    h2u_ref[...] = h2u
    h2i_ref[...] = h2i
    t2_ref[0, ...] = jnp.dot(h2u, w2f_ref[...], preferred_element_type=jnp.float32) + b2f_ref[...]
    t2_ref[1, ...] = jnp.dot(h2i, w2bb_ref[...], preferred_element_type=jnp.float32) + b2bb_ref[...]
    t2_ref[2, ...] = jnp.dot(h2u, w2b_ref[...], preferred_element_type=jnp.float32) + b2b_ref[...]


def _stage_c(sf, sb, cntf, cntb, cntbb, b1bb, w2f_p, b2f_p, w2bb_p, b2bb_p,
             w2b_p, b2b_p):
    blk = 2000
    grid = NU // blk
    return pl.pallas_call(
        _stage_c_body,
        grid=(grid,),
        in_specs=[
            pl.BlockSpec((NC, blk, HF), lambda i: (0, i, 0)),
            pl.BlockSpec((NC, blk, HF), lambda i: (0, i, 0)),
            pl.BlockSpec((blk, 16), lambda i: (i, 0)),
            pl.BlockSpec((blk, 16), lambda i: (i, 0)),
            pl.BlockSpec((blk, 16), lambda i: (i, 0)),
            pl.BlockSpec((1, F), lambda i: (0, 0)),
            pl.BlockSpec((F, 16), lambda i: (0, 0)),
            pl.BlockSpec((1, 16), lambda i: (0, 0)),
            pl.BlockSpec((F, 16), lambda i: (0, 0)),
            pl.BlockSpec((1, 16), lambda i: (0, 0)),
            pl.BlockSpec((F, 16), lambda i: (0, 0)),
            pl.BlockSpec((1, 16), lambda i: (0, 0)),
        ],
        out_specs=[
            pl.BlockSpec((blk, F), lambda i: (i, 0)),
            pl.BlockSpec((blk, F), lambda i: (i, 0)),
            pl.BlockSpec((3, blk, 16), lambda i: (0, i, 0)),
        ],
        out_shape=[
            jax.ShapeDtypeStruct((NU, F), jnp.float32),
            jax.ShapeDtypeStruct((NI, F), jnp.float32),
            jax.ShapeDtypeStruct((3, NU, 16), jnp.float32),
        ],
    )(sf, sb, cntf, cntb, cntbb, b1bb, w2f_p, b2f_p, w2bb_p, b2bb_p, w2b_p, b2b_p)


def _sc2_body(t2_hbm, src_hbm, dst_hbm, z16_hbm,
              s2_out, acc, idx_src, idx_dst, rows,
              gsem0, gsem1, ssem0, ssem1):
    cid = lax.axis_index("c")
    sid = lax.axis_index("s")
    row0 = sid * ROWS_PER_TILE2

    for q in range(3):
        pltpu.sync_copy(z16_hbm, acc.at[pl.ds(row0 + q * 640, 640)])
    plsc.subcore_barrier()

    pltpu.sync_copy(src_hbm.at[cid, sid], idx_src)
    pltpu.sync_copy(dst_hbm.at[cid, sid], idx_dst)

    _edge_loop(t2_hbm, idx_src, idx_dst, rows, (gsem0, gsem1),
               (ssem0, ssem1), acc, NCHUNK2, [])
    plsc.subcore_barrier()

    pltpu.sync_copy(acc.at[pl.ds(row0, ROWS_PER_TILE2)],
                    s2_out.at[cid, pl.ds(row0, ROWS_PER_TILE2)])


def _stage_d(t2, bsrc, bdst, z16):
    mesh = plsc.VectorSubcoreMesh(core_axis_name="c", subcore_axis_name="s",
                                  num_cores=NC, num_subcores=NS)
    f = pl.kernel(
        _sc2_body,
        out_type=jax.ShapeDtypeStruct((NC, PAD2, 16), jnp.float32),
        mesh=mesh,
        scratch_types=[
            pltpu.VMEM_SHARED((PAD2, 16), jnp.float32),
            pltpu.VMEM((NCHUNK2, K2), jnp.int32),
            pltpu.VMEM((NCHUNK2, K2), jnp.int32),
            pltpu.VMEM((2, K2, 16), jnp.float32),
            pltpu.SemaphoreType.DMA,
            pltpu.SemaphoreType.DMA,
            pltpu.SemaphoreType.DMA,
            pltpu.SemaphoreType.DMA,
        ],
        compiler_params=pltpu.CompilerParams(use_tc_tiling_on_sc=False),
    )
    return f(t2, bsrc, bdst, z16)


def _stage_e_body(s2_ref, cntf_ref, cntb_ref, cntbb_ref, ou_ref, oi_ref):
    sf = s2_ref[0, 0] + s2_ref[1, 0]
    sbb = s2_ref[0, 1] + s2_ref[1, 1]
    sb = s2_ref[0, 2] + s2_ref[1, 2]
    ou_ref[...] = (sf / jnp.maximum(cntf_ref[:, 0:1], 1.0)
                   + sbb / jnp.maximum(cntbb_ref[:, 0:1], 1.0))
    oi_ref[...] = sb / jnp.maximum(cntb_ref[:, 0:1], 1.0)


def _stage_e(s2, cntf, cntb, cntbb):
    blk = 2000
    grid = NU // blk
    return pl.pallas_call(
        _stage_e_body,
        grid=(grid,),
        in_specs=[
            pl.BlockSpec((NC, 3, blk, 16), lambda i: (0, 0, i, 0)),
            pl.BlockSpec((blk, 16), lambda i: (i, 0)),
            pl.BlockSpec((blk, 16), lambda i: (i, 0)),
            pl.BlockSpec((blk, 16), lambda i: (i, 0)),
        ],
        out_specs=[
            pl.BlockSpec((blk, 16), lambda i: (i, 0)),
            pl.BlockSpec((blk, 16), lambda i: (i, 0)),
        ],
        out_shape=[
            jax.ShapeDtypeStruct((NU, 16), jnp.float32),
            jax.ShapeDtypeStruct((NI, 16), jnp.float32),
        ],
    )(s2, cntf, cntb, cntbb)


@jax.jit
def kernel(node_feature, follows_src, follows_dst, buys_src, buys_dst,
           bought_by_src, bought_by_dst,
           W1_follows, b1_follows, W1_buys, b1_buys, W1_bought_by, b1_bought_by,
           W2_follows, b2_follows, W2_buys, b2_buys, W2_bought_by, b2_bought_by):
    srcf2 = jnp.stack([follows_src, follows_src + NU]).reshape(NC, NS, NCHUNK1, K1)
    srcb2 = jnp.stack([buys_src, buys_src + NU]).reshape(NC, NS, NCHUNK1, K1)
    dstf = follows_dst.reshape(NS, NCHUNK1, K1)
    dstb = buys_dst.reshape(NS, NCHUNK1, K1)
    dstbb = bought_by_dst.reshape(NS, NCHUNK1, K1)
    bsrc = jnp.concatenate([follows_src, bought_by_src + NU,
                            buys_src + 2 * NU]).reshape(NC, NS, NCHUNK2, K2)
    bdst = jnp.concatenate([follows_dst, bought_by_dst + NU,
                            buys_dst + 2 * NU]).reshape(NC, NS, NCHUNK2, K2)
    z128 = jnp.zeros((ROWS_PER_TILE1, HF), jnp.float32)
    z16 = jnp.zeros((640, 16), jnp.float32)
    ones16 = jnp.ones((K1, 16), jnp.float32)

    def pad16(w, b):
        wp = jnp.zeros((F, 16), jnp.float32).at[:, :2].set(w)
        bp = jnp.zeros((1, 16), jnp.float32).at[0, :2].set(b)
        return wp, bp

    w2f_p, b2f_p = pad16(W2_follows, b2_follows)
    w2bb_p, b2bb_p = pad16(W2_bought_by, b2_bought_by)
    w2b_p, b2b_p = pad16(W2_buys, b2_buys)

    tf, tb = _stage_a(node_feature, W1_follows, b1_follows, W1_buys, b1_buys)

    sf, sb, cntf, cntb, cntbb = _stage_b(
        tf.reshape(NC * NU, HF), tb.reshape(NC * NI, HF),
        srcf2, dstf, srcb2, dstb, dstbb, z128, z16, ones16)

    sf = sf[:, :NU]
    sb = sb[:, :NU]
    cntf = cntf[:NU]
    cntb = cntb[:NU]
    cntbb = cntbb[:NU]

    h2u, h2i, t2 = _stage_c(sf, sb, cntf, cntb, cntbb,
                            b1_bought_by.reshape(1, F),
                            w2f_p, b2f_p, w2bb_p, b2bb_p, w2b_p, b2b_p)

    s2 = _stage_d(t2.reshape(3 * NU, 16), bsrc, bdst, z16)

    ou, oi = _stage_e(s2[:, :3 * NU].reshape(NC, 3, NU, 16), cntf, cntb, cntbb)

    return ou[:, :2], oi[:, :2], h2u, h2i


# speedup vs baseline: 1.3689x; 1.0100x over previous
import jax
import jax.numpy as jnp
from jax import lax
from jax.experimental import pallas as pl
from jax.experimental.pallas import tpu as pltpu
from jax.experimental.pallas import tpu_sc as plsc

NU = 10000
NI = 10000
E = 160000
F = 256
HF = 128
NC = 2
NS = 16

K1 = 80
CPB1 = 42
NBLK1 = 3
EP1 = NS * NBLK1 * CPB1 * K1

K2 = 120
NCHUNK2 = (3 * E // (NC * NS)) // K2

PAD1 = 10240
PAD2 = 30720
ROWS_PER_TILE1 = PAD1 // NS
ROWS_PER_TILE2 = PAD2 // NS
DPAD1 = NU


def _stage_a_body(x_ref, w1f_ref, b1f_ref, w1b_ref, b1b_ref, tf_ref, tb_ref):
    x = x_ref[...]
    whf = jnp.dot(x, w1f_ref[...], preferred_element_type=jnp.float32) + b1f_ref[...]
    whb = jnp.dot(x, w1b_ref[...], preferred_element_type=jnp.float32) + b1b_ref[...]
    tf_ref[0, ...] = whf[:, :HF]
    tf_ref[1, ...] = whf[:, HF:]
    tb_ref[0, ...] = whb[:, :HF]
    tb_ref[1, ...] = whb[:, HF:]


def _stage_a(x, w1f, b1f, w1b, b1b):
    blk = 2000
    grid = NU // blk
    return pl.pallas_call(
        _stage_a_body,
        grid=(grid,),
        in_specs=[
            pl.BlockSpec((blk, F), lambda i: (i, 0)),
            pl.BlockSpec((F, F), lambda i: (0, 0)),
            pl.BlockSpec((1, F), lambda i: (0, 0)),
            pl.BlockSpec((F, F), lambda i: (0, 0)),
            pl.BlockSpec((1, F), lambda i: (0, 0)),
        ],
        out_specs=[
            pl.BlockSpec((NC, blk, HF), lambda i: (0, i, 0)),
            pl.BlockSpec((NC, blk, HF), lambda i: (0, i, 0)),
        ],
        out_shape=[
            jax.ShapeDtypeStruct((NC, NU, HF), jnp.float32),
            jax.ShapeDtypeStruct((NC, NU, HF), jnp.float32),
        ],
    )(x, w1f, b1f.reshape(1, F), w1b, b1b.reshape(1, F))


def _sc1_body(tf_hbm, tb_hbm, srcf_hbm, dstf_hbm, srcb_hbm, dstb_hbm,
              dstbb_hbm, z128_hbm, z16_hbm, ones_hbm,
              sf_out, sb_out, cntf_out, cntb_out, cntbb_out,
              acc, cnt, isrc_a, idst_a, isrc_b, idst_b, rows, ones_v,
              gsem0, gsem1, isem, bbsem):
    cid = lax.axis_index("c")
    sid = lax.axis_index("s")
    row0 = sid * ROWS_PER_TILE1
    gsems = (gsem0, gsem1)

    pltpu.sync_copy(ones_hbm, ones_v)

    def run_phase(tab, src2, dst, s_out, cnt_zero_pred, count_pred,
                  cnt_flushes):
        pltpu.sync_copy(z128_hbm, acc.at[pl.ds(row0, ROWS_PER_TILE1)])

        @pl.when(cnt_zero_pred)
        def _():
            pltpu.sync_copy(z16_hbm, cnt.at[pl.ds(row0, ROWS_PER_TILE1)])

        plsc.subcore_barrier()

        def g_issue(ib, r, p):
            pltpu.async_copy(tab.at[ib.at[r]], rows.at[p], gsems[p])

        def g_wait(p):
            pltpu.make_async_copy(tab.at[isrc_a.at[0]], rows.at[p],
                                  gsems[p]).wait()

        def consume(idb, r, p):
            pltpu.sync_copy(rows.at[p], acc.at[idb.at[r]], add=True)

            @pl.when(count_pred)
            def _():
                pltpu.sync_copy(ones_v, cnt.at[idb.at[r]], add=True)

        pltpu.sync_copy(src2.at[cid, sid, 0], isrc_a)
        pltpu.sync_copy(dst.at[sid, 0], idst_a)

        for b in range(NBLK1):
            ibs, idb = (isrc_a, idst_a) if b % 2 == 0 else (isrc_b, idst_b)
            nbs, ndb = (isrc_b, idst_b) if b % 2 == 0 else (isrc_a, idst_a)
            if b + 1 < NBLK1:
                pltpu.async_copy(src2.at[cid, sid, b + 1], nbs, isem)
                pltpu.async_copy(dst.at[sid, b + 1], ndb, isem)

            g_issue(ibs, 0, 0)

            def pair(h, _, ibs=ibs, idb=idb):
                r0 = 2 * h
                r1 = 2 * h + 1
                g_issue(ibs, r1, 1)
                g_wait(0)
                consume(idb, r0, 0)

                @pl.when(h < CPB1 // 2 - 1)
                def _():
                    g_issue(ibs, r0 + 2, 0)

                g_wait(1)
                consume(idb, r1, 1)
                return 0

            lax.fori_loop(0, CPB1 // 2, pair, 0)

            if b + 1 < NBLK1:
                pltpu.make_async_copy(src2.at[cid, sid, 0], nbs, isem).wait()
                pltpu.make_async_copy(dst.at[sid, 0], ndb, isem).wait()

        plsc.subcore_barrier()

        pltpu.sync_copy(acc.at[pl.ds(row0, ROWS_PER_TILE1)],
                        s_out.at[cid, pl.ds(row0, ROWS_PER_TILE1)])

        for pred, cnt_out in cnt_flushes:
            @pl.when(pred)
            def _():
                pltpu.sync_copy(cnt.at[pl.ds(row0, ROWS_PER_TILE1)],
                                cnt_out.at[pl.ds(row0, ROWS_PER_TILE1)])

    run_phase(tf_hbm, srcf_hbm, dstf_hbm, sf_out,
              cnt_zero_pred=(cid >= 0), count_pred=(cid == 0),
              cnt_flushes=[(cid == 0, cntf_out)])

    @pl.when(cid == 1)
    def _():
        for b in range(NBLK1):
            pltpu.sync_copy(dstbb_hbm.at[sid, b], idst_a)

            def bb_group(h, _):
                for j in range(CPB1 // 2):
                    pltpu.async_copy(ones_v,
                                     cnt.at[idst_a.at[h * (CPB1 // 2) + j]],
                                     bbsem, add=True)
                for j in range(CPB1 // 2):
                    pltpu.make_async_copy(ones_v, cnt.at[idst_a.at[0]],
                                          bbsem).wait()
                return 0

            lax.fori_loop(0, 2, bb_group, 0)

    plsc.subcore_barrier()

    @pl.when(cid == 1)
    def _():
        pltpu.sync_copy(cnt.at[pl.ds(row0, ROWS_PER_TILE1)],
                        cntbb_out.at[pl.ds(row0, ROWS_PER_TILE1)])

    run_phase(tb_hbm, srcb_hbm, dstb_hbm, sb_out,
              cnt_zero_pred=(cid == 0), count_pred=(cid == 0),
              cnt_flushes=[(cid == 0, cntb_out)])


def _stage_b(tf, tb, srcf2, dstf, srcb2, dstb, dstbb, z128, z16, ones16):
    mesh = plsc.VectorSubcoreMesh(core_axis_name="c", subcore_axis_name="s",
                                  num_cores=NC, num_subcores=NS)
    f = pl.kernel(
        _sc1_body,
        out_type=[
            jax.ShapeDtypeStruct((NC, PAD1, HF), jnp.float32),
            jax.ShapeDtypeStruct((NC, PAD1, HF), jnp.float32),
            jax.ShapeDtypeStruct((PAD1, 16), jnp.float32),
            jax.ShapeDtypeStruct((PAD1, 16), jnp.float32),
            jax.ShapeDtypeStruct((PAD1, 16), jnp.float32),
        ],
        mesh=mesh,
        scratch_types=[
            pltpu.VMEM_SHARED((PAD1, HF), jnp.float32),
            pltpu.VMEM_SHARED((PAD1, 16), jnp.float32),
            pltpu.VMEM((CPB1, K1), jnp.int32),
            pltpu.VMEM((CPB1, K1), jnp.int32),
            pltpu.VMEM((CPB1, K1), jnp.int32),
            pltpu.VMEM((CPB1, K1), jnp.int32),
            pltpu.VMEM((2, K1, HF), jnp.float32),
            pltpu.VMEM((K1, 16), jnp.float32),
            pltpu.SemaphoreType.DMA,
            pltpu.SemaphoreType.DMA,
            pltpu.SemaphoreType.DMA,
            pltpu.SemaphoreType.DMA,
        ],
        compiler_params=pltpu.CompilerParams(use_tc_tiling_on_sc=False),
    )
    return f(tf, tb, srcf2, dstf, srcb2, dstb, dstbb, z128, z16, ones16)


def _stage_c_body(sf_ref, sb_ref, cntf_ref, cntb_ref, cntbb_ref, b1bb_ref,
                  w2f_ref, b2f_ref, w2bb_ref, b2bb_ref, w2b_ref, b2b_ref,
                  h2u_ref, h2i_ref, t2_ref):
    cf = jnp.maximum(cntf_ref[:, 0:1], 1.0)
    cb = jnp.maximum(cntb_ref[:, 0:1], 1.0)
    ibb = (cntbb_ref[:, 0:1] > 0.0).astype(jnp.float32)
    h1u = jnp.concatenate([sf_ref[0], sf_ref[1]], axis=1) / cf + b1bb_ref[...] * ibb
    h1i = jnp.concatenate([sb_ref[0], sb_ref[1]], axis=1) / cb
    h2u = jnp.where(h1u >= 0.0, h1u, 0.01 * h1u)
    h2i = jnp.where(h1i >= 0.0, h1i, 0.01 * h1i)
    h2u_ref[...] = h2u
    h2i_ref[...] = h2i
    t2_ref[0, ...] = jnp.dot(h2u, w2f_ref[...], preferred_element_type=jnp.float32) + b2f_ref[...]
    t2_ref[1, ...] = jnp.dot(h2i, w2bb_ref[...], preferred_element_type=jnp.float32) + b2bb_ref[...]
    t2_ref[2, ...] = jnp.dot(h2u, w2b_ref[...], preferred_element_type=jnp.float32) + b2b_ref[...]


def _stage_c(sf, sb, cntf, cntb, cntbb, b1bb, w2f_p, b2f_p, w2bb_p, b2bb_p,
             w2b_p, b2b_p):
    blk = 2000
    grid = NU // blk
    return pl.pallas_call(
        _stage_c_body,
        grid=(grid,),
        in_specs=[
            pl.BlockSpec((NC, blk, HF), lambda i: (0, i, 0)),
            pl.BlockSpec((NC, blk, HF), lambda i: (0, i, 0)),
            pl.BlockSpec((blk, 16), lambda i: (i, 0)),
            pl.BlockSpec((blk, 16), lambda i: (i, 0)),
            pl.BlockSpec((blk, 16), lambda i: (i, 0)),
            pl.BlockSpec((1, F), lambda i: (0, 0)),
            pl.BlockSpec((F, 16), lambda i: (0, 0)),
            pl.BlockSpec((1, 16), lambda i: (0, 0)),
            pl.BlockSpec((F, 16), lambda i: (0, 0)),
            pl.BlockSpec((1, 16), lambda i: (0, 0)),
            pl.BlockSpec((F, 16), lambda i: (0, 0)),
            pl.BlockSpec((1, 16), lambda i: (0, 0)),
        ],
        out_specs=[
            pl.BlockSpec((blk, F), lambda i: (i, 0)),
            pl.BlockSpec((blk, F), lambda i: (i, 0)),
            pl.BlockSpec((3, blk, 16), lambda i: (0, i, 0)),
        ],
        out_shape=[
            jax.ShapeDtypeStruct((NU, F), jnp.float32),
            jax.ShapeDtypeStruct((NI, F), jnp.float32),
            jax.ShapeDtypeStruct((3, NU, 16), jnp.float32),
        ],
    )(sf, sb, cntf, cntb, cntbb, b1bb, w2f_p, b2f_p, w2bb_p, b2bb_p, w2b_p, b2b_p)


def _sc2_body(t2_hbm, src_hbm, dst_hbm, z16_hbm,
              s2_out, acc, idx_src, idx_dst, rows, gsem0, gsem1):
    cid = lax.axis_index("c")
    sid = lax.axis_index("s")
    row0 = sid * ROWS_PER_TILE2
    gsems = (gsem0, gsem1)

    for q in range(3):
        pltpu.sync_copy(z16_hbm, acc.at[pl.ds(row0 + q * 640, 640)])
    plsc.subcore_barrier()

    pltpu.sync_copy(src_hbm.at[cid, sid], idx_src)
    pltpu.sync_copy(dst_hbm.at[cid, sid], idx_dst)

    def g_issue(c, p):
        pltpu.async_copy(t2_hbm.at[idx_src.at[c]], rows.at[p], gsems[p])

    def g_wait(p):
        pltpu.make_async_copy(t2_hbm.at[idx_src.at[0]], rows.at[p],
                              gsems[p]).wait()

    def consume(c, p):
        pltpu.sync_copy(rows.at[p], acc.at[idx_dst.at[c]], add=True)

    g_issue(0, 0)

    def pair(h, _):
        c0 = 2 * h
        c1 = 2 * h + 1

        @pl.when(c1 < NCHUNK2)
        def _():
            g_issue(c1, 1)

        g_wait(0)
        consume(c0, 0)

        @pl.when(c0 + 2 < NCHUNK2)
        def _():
            g_issue(c0 + 2, 0)

        @pl.when(c1 < NCHUNK2)
        def _():
            g_wait(1)
            consume(c1, 1)

        return 0

    lax.fori_loop(0, (NCHUNK2 + 1) // 2, pair, 0)
    plsc.subcore_barrier()

    pltpu.sync_copy(acc.at[pl.ds(row0, ROWS_PER_TILE2)],
                    s2_out.at[cid, pl.ds(row0, ROWS_PER_TILE2)])


def _stage_d(t2, bsrc, bdst, z16):
    mesh = plsc.VectorSubcoreMesh(core_axis_name="c", subcore_axis_name="s",
                                  num_cores=NC, num_subcores=NS)
    f = pl.kernel(
        _sc2_body,
        out_type=jax.ShapeDtypeStruct((NC, PAD2, 16), jnp.float32),
        mesh=mesh,
        scratch_types=[
            pltpu.VMEM_SHARED((PAD2, 16), jnp.float32),
            pltpu.VMEM((NCHUNK2, K2), jnp.int32),
            pltpu.VMEM((NCHUNK2, K2), jnp.int32),
            pltpu.VMEM((2, K2, 16), jnp.float32),
            pltpu.SemaphoreType.DMA,
            pltpu.SemaphoreType.DMA,
        ],
        compiler_params=pltpu.CompilerParams(use_tc_tiling_on_sc=False),
    )
    return f(t2, bsrc, bdst, z16)


def _stage_e_body(s2_ref, cntf_ref, cntb_ref, cntbb_ref, ou_ref, oi_ref):
    sf = s2_ref[0, 0] + s2_ref[1, 0]
    sbb = s2_ref[0, 1] + s2_ref[1, 1]
    sb = s2_ref[0, 2] + s2_ref[1, 2]
    ou_ref[...] = (sf / jnp.maximum(cntf_ref[:, 0:1], 1.0)
                   + sbb / jnp.maximum(cntbb_ref[:, 0:1], 1.0))
    oi_ref[...] = sb / jnp.maximum(cntb_ref[:, 0:1], 1.0)


def _stage_e(s2, cntf, cntb, cntbb):
    blk = 2000
    grid = NU // blk
    return pl.pallas_call(
        _stage_e_body,
        grid=(grid,),
        in_specs=[
            pl.BlockSpec((NC, 3, blk, 16), lambda i: (0, 0, i, 0)),
            pl.BlockSpec((blk, 16), lambda i: (i, 0)),
            pl.BlockSpec((blk, 16), lambda i: (i, 0)),
            pl.BlockSpec((blk, 16), lambda i: (i, 0)),
        ],
        out_specs=[
            pl.BlockSpec((blk, 16), lambda i: (i, 0)),
            pl.BlockSpec((blk, 16), lambda i: (i, 0)),
        ],
        out_shape=[
            jax.ShapeDtypeStruct((NU, 16), jnp.float32),
            jax.ShapeDtypeStruct((NI, 16), jnp.float32),
        ],
    )(s2, cntf, cntb, cntbb)


@jax.jit
def kernel(node_feature, follows_src, follows_dst, buys_src, buys_dst,
           bought_by_src, bought_by_dst,
           W1_follows, b1_follows, W1_buys, b1_buys, W1_bought_by, b1_bought_by,
           W2_follows, b2_follows, W2_buys, b2_buys, W2_bought_by, b2_bought_by):
    npad = EP1 - E
    zpad = jnp.zeros((npad,), jnp.int32)
    dpad = jnp.full((npad,), DPAD1, jnp.int32)

    def pad_edges(src, dst):
        return jnp.concatenate([src, zpad]), jnp.concatenate([dst, dpad])

    fsrc, fdst = pad_edges(follows_src, follows_dst)
    busrc, budst = pad_edges(buys_src, buys_dst)
    _, bbdst = pad_edges(bought_by_src, bought_by_dst)

    srcf2 = jnp.stack([fsrc, fsrc + NU]).reshape(NC, NS, NBLK1, CPB1, K1)
    srcb2 = jnp.stack([busrc, busrc + NU]).reshape(NC, NS, NBLK1, CPB1, K1)
    dstf = fdst.reshape(NS, NBLK1, CPB1, K1)
    dstb = budst.reshape(NS, NBLK1, CPB1, K1)
    dstbb = bbdst.reshape(NS, NBLK1, CPB1, K1)

    big_src = jnp.concatenate([follows_src, bought_by_src + NU,
                               buys_src + 2 * NU]).reshape(NC, NS, NCHUNK2, K2)
    big_dst = jnp.concatenate([follows_dst, bought_by_dst + NU,
                               buys_dst + 2 * NU]).reshape(NC, NS, NCHUNK2, K2)

    z128 = jnp.zeros((ROWS_PER_TILE1, HF), jnp.float32)
    z16 = jnp.zeros((640, 16), jnp.float32)
    ones16 = jnp.ones((K1, 16), jnp.float32)

    def pad16(w, b):
        wp = jnp.zeros((F, 16), jnp.float32).at[:, :2].set(w)
        bp = jnp.zeros((1, 16), jnp.float32).at[0, :2].set(b)
        return wp, bp

    w2f_p, b2f_p = pad16(W2_follows, b2_follows)
    w2bb_p, b2bb_p = pad16(W2_bought_by, b2_bought_by)
    w2b_p, b2b_p = pad16(W2_buys, b2_buys)

    tf, tb = _stage_a(node_feature, W1_follows, b1_follows, W1_buys, b1_buys)

    sf, sb, cntf, cntb, cntbb = _stage_b(
        tf.reshape(NC * NU, HF), tb.reshape(NC * NI, HF),
        srcf2, dstf, srcb2, dstb, dstbb, z128, z16, ones16)

    h2u, h2i, t2 = _stage_c(sf, sb, cntf, cntb, cntbb,
                            b1_bought_by.reshape(1, F),
                            w2f_p, b2f_p, w2bb_p, b2bb_p, w2b_p, b2b_p)

    s2 = _stage_d(t2.reshape(3 * NU, 16), big_src, big_dst, z16)

    ou, oi = _stage_e(s2[:, :3 * NU].reshape(NC, 3, NU, 16), cntf, cntb, cntbb)

    return ou[:, :2], oi[:, :2], h2u, h2i
